# Initial kernel scaffold; baseline (speedup 1.0000x reference)
import jax
import jax.numpy as jnp
from jax.experimental import pallas as pl


def kernel(queries, points):
    raise NotImplementedError("write your pallas kernel here")


# speedup vs baseline: 1.8187x; 1.8187x over previous
import jax
import jax.numpy as jnp
from jax import lax
from jax.experimental import pallas as pl

_QB = 256


def _nn_body(q_ref, pt_ref, out_ref):
    P = pt_ref.shape[2]
    q = q_ref[0]
    pt = pt_ref[0]
    qp = lax.dot_general(
        q.astype(jnp.bfloat16), pt.astype(jnp.bfloat16),
        dimension_numbers=(((1,), (0,)), ((), ())),
        preferred_element_type=jnp.float32)
    q0 = q[:, 0:1]
    q1 = q[:, 1:2]
    q2 = q[:, 2:3]
    qq = q0 * q0 + q1 * q1 + q2 * q2
    px = pt[0:1, :]
    py = pt[1:2, :]
    pz = pt[2:3, :]
    pp = px * px + py * py + pz * pz
    key = (qq - 2.0 * qp) + pp
    m = jnp.min(key, axis=1, keepdims=True)
    iota = lax.broadcasted_iota(jnp.int32, (_QB, P), 1)
    idx = jnp.min(jnp.where(key == m, iota, P), axis=1)
    out_ref[0, 0, :] = idx


def kernel(queries, points):
    B, Q, _ = queries.shape
    P = points.shape[1]
    pt = points.transpose(0, 2, 1)
    nq = Q // _QB
    out = pl.pallas_call(
        _nn_body,
        grid=(B, nq),
        in_specs=[
            pl.BlockSpec((1, _QB, 3), lambda b, i: (b, i, 0)),
            pl.BlockSpec((1, 3, P), lambda b, i: (b, 0, 0)),
        ],
        out_specs=pl.BlockSpec((1, 1, _QB), lambda b, i: (b * nq + i, 0, 0)),
        out_shape=jax.ShapeDtypeStruct((B * nq, 1, _QB), jnp.int32),
    )(queries, pt)
    return out.reshape(B, Q).astype(jnp.int64)


# speedup vs baseline: 2.1356x; 1.1742x over previous
import jax
import jax.numpy as jnp
from jax import lax
from jax.experimental import pallas as pl

_QB = 256
_C = 512


def _nn_body(q_ref, pt_ref, out_ref):
    P = pt_ref.shape[2]
    n_chunks = P // _C
    q = q_ref[0]
    qbm2 = q.astype(jnp.bfloat16) * jnp.bfloat16(-2.0)
    q0 = q[:, 0:1]
    q1 = q[:, 1:2]
    q2 = q[:, 2:3]
    qq = q0 * q0 + q1 * q1 + q2 * q2

    def chunk_key(c):
        pt = pt_ref[0, :, pl.ds(c * _C, _C)]
        qp2 = lax.dot_general(
            qbm2, pt.astype(jnp.bfloat16),
            dimension_numbers=(((1,), (0,)), ((), ())),
            preferred_element_type=jnp.float32)
        px = pt[0:1, :]
        py = pt[1:2, :]
        pz = pt[2:3, :]
        pp = px * px + py * py + pz * pz
        return (qq + qp2) + pp

    nodes = [(chunk_key(c), c) for c in range(n_chunks)]
    while len(nodes) > 1:
        nxt = []
        for j in range(0, len(nodes), 2):
            (ka, ia), (kb, ib) = nodes[j], nodes[j + 1]
            lt = kb < ka
            k = jnp.where(lt, kb, ka)
            if isinstance(ia, int) and isinstance(ib, int):
                i = jnp.where(lt, ib, ia)
            else:
                i = jnp.where(lt, ib, ia)
            nxt.append((k, i))
        nodes = nxt
    best, bidx = nodes[0]

    m = jnp.min(best, axis=1, keepdims=True)
    lane = lax.broadcasted_iota(jnp.int32, (_QB, _C), 1)
    full = bidx * _C + lane
    idx = jnp.min(jnp.where(best == m, full, P), axis=1)
    out_ref[0, 0, :] = idx


def kernel(queries, points):
    B, Q, _ = queries.shape
    P = points.shape[1]
    pt = points.transpose(0, 2, 1)
    nq = Q // _QB
    out = pl.pallas_call(
        _nn_body,
        grid=(B, nq),
        in_specs=[
            pl.BlockSpec((1, _QB, 3), lambda b, i: (b, i, 0)),
            pl.BlockSpec((1, 3, P), lambda b, i: (b, 0, 0)),
        ],
        out_specs=pl.BlockSpec((1, 1, _QB), lambda b, i: (b * nq + i, 0, 0)),
        out_shape=jax.ShapeDtypeStruct((B * nq, 1, _QB), jnp.int32),
    )(queries, pt)
    return out.reshape(B, Q).astype(jnp.int64)
